# Initial kernel scaffold; baseline (speedup 1.0000x reference)
import jax
import jax.numpy as jnp
from jax.experimental import pallas as pl


def kernel(x, table):
    raise NotImplementedError("write your pallas kernel here")


# speedup vs baseline: 4.6628x; 4.6628x over previous
import functools

import jax
import jax.numpy as jnp
from jax import lax
from jax.experimental import pallas as pl
from jax.experimental.pallas import tpu as pltpu
from jax.experimental.pallas import tpu_sc as plsc

NC = 2
NS = 16
NW = NC * NS
GW = 4
NBUF = 2 * GW


@jax.jit
def _gather(x, table):
    nbatch, seq = x.shape
    dim = table.shape[1]
    nb = nbatch // NW
    ngroup = nb // GW
    mesh = plsc.VectorSubcoreMesh(core_axis_name="c", subcore_axis_name="s")

    @functools.partial(
        pl.kernel,
        mesh=mesh,
        out_type=jax.ShapeDtypeStruct((nbatch, seq, dim), jnp.float32),
        compiler_params=pltpu.CompilerParams(use_tc_tiling_on_sc=False),
        scratch_types=[
            pltpu.VMEM((nb, seq), jnp.int32),
            pltpu.VMEM((NBUF, seq, dim), jnp.float32),
        ]
        + [pltpu.SemaphoreType.DMA] * NBUF,
    )
    def body(x_hbm, table_hbm, out_hbm, idx_v, rows_v, *gsems):
        wid = lax.axis_index("s") * NC + lax.axis_index("c")
        b0 = wid * nb
        pltpu.sync_copy(x_hbm.at[pl.ds(b0, nb)], idx_v)

        def fire(c, b):
            pltpu.async_copy(
                table_hbm.at[idx_v.at[c]], rows_v.at[b], gsems[b]
            )

        def drain(c, b):
            pltpu.make_async_copy(
                table_hbm.at[idx_v.at[c]], rows_v.at[b], gsems[b]
            ).wait()

        for b in range(NBUF):
            fire(b, b)

        def step(i, carry):
            for p in range(2):
                g = 2 * i + p
                c0 = GW * g
                for j in range(GW):
                    drain(c0 + j, GW * p + j)
                pltpu.sync_copy(
                    rows_v.at[pl.ds(GW * p, GW)], out_hbm.at[pl.ds(b0 + c0, GW)]
                )
                for j in range(GW):
                    fire(c0 + NBUF + j, GW * p + j)
            return carry

        lax.fori_loop(0, ngroup // 2 - 1, step, 0)
        for p in range(2):
            c0 = GW * (ngroup - 2 + p)
            for j in range(GW):
                drain(c0 + j, GW * p + j)
            pltpu.sync_copy(
                rows_v.at[pl.ds(GW * p, GW)], out_hbm.at[pl.ds(b0 + c0, GW)]
            )

    return body(x, table)


def kernel(x, table):
    assert x.shape[0] % (NW * GW * 2) == 0
    return _gather(x.astype(jnp.int32), table)
